# baseline (speedup 1.0000x reference)
import functools

import jax
import jax.numpy as jnp
from jax.experimental import pallas as pl
from jax.experimental.pallas import tpu as pltpu

_BF16 = jnp.bfloat16
_VMEM_LIMIT = 48 * 1024 * 1024


def _cparams():
    return pltpu.CompilerParams(dimension_semantics=("parallel",),
                                vmem_limit_bytes=_VMEM_LIMIT)


def _evenrows(v, off, n):
    return v[off:off + 2 * n].reshape(n, 2, v.shape[1], v.shape[2])[:, 0]


def _colsplit(v):
    r, c, ch = v.shape
    v2 = v.reshape(r, c // 2, 2, ch)
    return v2[:, :, 0], v2[:, :, 1]


def _bneck_compute(cur, wr, stride, has_down, mid_ref, pk_ref):
    H, W, cin = cur.shape
    w1, b1, w2, b2, w3, b3 = wr[:6]
    mid = w1.shape[-1]
    ho, wo = H // stride, W // stride
    M = ho * wo

    h1 = jnp.dot(cur.reshape(H * W, cin), w1[...],
                 preferred_element_type=jnp.float32) + b1[...]
    mid_ref[1:H + 1, 1:W + 1, :] = (
        jnp.maximum(h1, 0.0).astype(_BF16).reshape(H, W, mid))
    zr = jnp.zeros((1, W + 2, mid), _BF16)
    mid_ref[0:1, 0:W + 2] = zr
    mid_ref[H + 1:H + 2, 0:W + 2] = zr
    zc = jnp.zeros((H + 2, 1, mid), _BF16)
    mid_ref[0:H + 2, 0:1] = zc
    mid_ref[0:H + 2, W + 1:W + 2] = zc
    mp = mid_ref[0:H + 2, 0:W + 2, :]

    acc2 = jnp.zeros((M, mid), jnp.float32)
    if stride == 1:
        for kj in range(3):
            pk_ref[0:H + 2, 0:W, kj * mid:(kj + 1) * mid] = mp[:, kj:kj + W]
        for ki in range(3):
            a = pk_ref[ki:ki + H, 0:W, :].reshape(M, 3 * mid)
            acc2 = acc2 + jnp.dot(a, w2[ki * 3 * mid:(ki + 1) * 3 * mid, :],
                                  preferred_element_type=jnp.float32)
    else:
        ce, co = _colsplit(mp)
        picks = ((ce, 0), (co, 0), (ce, 1))
        for kj in range(3):
            csel, j0 = picks[kj]
            pk_ref[0:H + 2, 0:wo, kj * mid:(kj + 1) * mid] = csel[:, j0:j0 + wo]
        for ki in range(3):
            a = pk_ref[ki:ki + 2 * ho, 0:wo, :]
            a = a.reshape(ho, 2, wo, 3 * mid)[:, 0].reshape(M, 3 * mid)
            acc2 = acc2 + jnp.dot(a, w2[ki * 3 * mid:(ki + 1) * 3 * mid, :],
                                  preferred_element_type=jnp.float32)
    h2 = jnp.maximum(acc2 + b2[...], 0.0).astype(_BF16)

    h3 = jnp.dot(h2, w3[...], preferred_element_type=jnp.float32) + b3[...]
    if has_down:
        wd, bd = wr[6:8]
        if stride == 1:
            xs = cur
        else:
            xe, _ = _colsplit(cur)
            xs = _evenrows(xe, 0, ho)
        idn = jnp.dot(xs.reshape(M, cin), wd[...],
                      preferred_element_type=jnp.float32) + bd[...]
        idn = idn.astype(_BF16).astype(jnp.float32)
    else:
        idn = cur.reshape(M, cin).astype(jnp.float32)
    out = jnp.maximum(h3 + idn, 0.0)
    return out.astype(_BF16).reshape(ho, wo, out.shape[-1])


def _layer_kernel(*refs, cfg, nw):
    x_ref = refs[0]
    wrefs = refs[1:1 + nw]
    o_ref = refs[1 + nw]
    mid_ref, pk_ref = refs[2 + nw:]
    cur = x_ref[0]
    i = 0
    for stride, has_down in cfg:
        k = 8 if has_down else 6
        cur = _bneck_compute(cur, wrefs[i:i + k], stride, has_down,
                             mid_ref, pk_ref)
        i += k
    o_ref[0] = cur


def _layer(x, blocks):
    N, H, W, cin0 = x.shape
    f32 = jnp.float32
    args = [x]
    in_specs = [pl.BlockSpec((1, H, W, cin0), lambda n: (n, 0, 0, 0))]
    cfg = []
    mid = blocks[0][2].shape[2]
    stride0 = blocks[0][-1]
    hl, wl = H // stride0, W // stride0
    cout = None
    for bp in blocks:
        stride = bp[-1]
        ws = bp[:-1]
        has_down = len(ws) == 8
        cfg.append((stride, has_down))
        w1, b1, w2, b2, w3, b3 = ws[:6]
        cin = w1.shape[2]
        cout = w3.shape[3]
        flat = [w1.reshape(cin, mid), b1.reshape(1, mid).astype(f32),
                w2.reshape(9 * mid, mid), b2.reshape(1, mid).astype(f32),
                w3.reshape(mid, cout), b3.reshape(1, cout).astype(f32)]
        if has_down:
            wd, bd = ws[6:8]
            flat += [wd.reshape(cin, cout), bd.reshape(1, cout).astype(f32)]
        for a in flat:
            args.append(a)
            in_specs.append(
                pl.BlockSpec(a.shape, lambda n, nd=a.ndim: (0,) * nd))
    nw = len(args) - 1
    scratch = [pltpu.VMEM((H + 2, W + 2, mid), _BF16),
               pltpu.VMEM((H + 2, wl, 3 * mid), _BF16)]
    return pl.pallas_call(
        functools.partial(_layer_kernel, cfg=tuple(cfg), nw=nw),
        out_shape=jax.ShapeDtypeStruct((N, hl, wl, cout), _BF16),
        grid=(N,),
        in_specs=in_specs,
        out_specs=pl.BlockSpec((1, hl, wl, cout), lambda n: (n, 0, 0, 0)),
        scratch_shapes=scratch,
        compiler_params=_cparams(),
    )(*args)


def _stem_kernel(x2_ref, w_ref, b_ref, o_ref, cv_ref, pk_ref, *, ho):
    x2 = x2_ref[0]
    for kj in range(4):
        pk_ref[0:ho + 3, 0:ho, kj * 16:(kj + 1) * 16] = x2[:, kj:kj + ho]
    acc = jnp.zeros((ho * ho, 64), jnp.float32)
    for ki in range(4):
        a = pk_ref[ki:ki + ho, 0:ho, :].reshape(ho * ho, 64)
        acc = acc + jnp.dot(a, w_ref[ki * 64:(ki + 1) * 64, :],
                            preferred_element_type=jnp.float32)
    h = acc + b_ref[...]
    cv_ref[1:ho + 1, 1:ho + 1, :] = (
        jnp.maximum(h, 0.0).astype(_BF16).reshape(ho, ho, 64))
    zr = jnp.zeros((1, ho + 2, 64), _BF16)
    cv_ref[0:1] = zr
    cv_ref[ho + 1:ho + 2] = zr
    zc = jnp.zeros((ho + 2, 1, 64), _BF16)
    cv_ref[:, 0:1] = zc
    cv_ref[:, ho + 1:ho + 2] = zc
    cv = cv_ref[...]
    hp = ho // 2
    cv2 = cv.reshape(hp + 1, 2, ho + 2, 64)
    re = cv2[:, 0]
    ro = cv2[:, 1]
    rm = jnp.maximum(jnp.maximum(re[0:hp], ro[0:hp]), re[1:hp + 1])
    ce, co = _colsplit(rm)
    m = jnp.maximum(jnp.maximum(ce[:, 0:hp], co[:, 0:hp]), ce[:, 1:hp + 1])
    o_ref[0] = m


def _stem_pool(x_nhwc4, stem_w, stem_bias):
    N, H, _, C = x_nhwc4.shape
    ho = H // 2
    hb = ho + 3
    xpad = jnp.pad(x_nhwc4, ((0, 0), (3, 3), (3, 3), (0, 0)))
    x2 = xpad.reshape(N, hb, 2, hb, 2, C).transpose(0, 1, 3, 2, 4, 5)
    x2 = x2.reshape(N, hb, hb, 4 * C)
    wp = jnp.pad(stem_w, ((0, 1), (0, 1), (0, 0), (0, 0)))
    w = wp.reshape(4, 2, 4, 2, C, 64).transpose(0, 2, 1, 3, 4, 5)
    w = w.reshape(16 * C * 4, 64)
    return pl.pallas_call(
        functools.partial(_stem_kernel, ho=ho),
        out_shape=jax.ShapeDtypeStruct((N, ho // 2, ho // 2, 64), _BF16),
        grid=(N,),
        in_specs=[
            pl.BlockSpec((1, hb, hb, 4 * C), lambda n: (n, 0, 0, 0)),
            pl.BlockSpec((16 * C * 4, 64), lambda n: (0, 0)),
            pl.BlockSpec((1, 64), lambda n: (0, 0)),
        ],
        out_specs=pl.BlockSpec((1, ho // 2, ho // 2, 64), lambda n: (n, 0, 0, 0)),
        scratch_shapes=[pltpu.VMEM((ho + 2, ho + 2, 64), _BF16),
                        pltpu.VMEM((ho + 3, ho, 16 * C), _BF16)],
        compiler_params=_cparams(),
    )(x2, w, stem_bias.reshape(1, 64).astype(jnp.float32))


def kernel(stem_w, stem_bias, l0b0_conv1_w, l0b0_conv1_bias, l0b0_conv2_w, l0b0_conv2_bias, l0b0_conv3_w, l0b0_conv3_bias, l0b0_down_w, l0b0_down_bias, l0b1_conv1_w, l0b1_conv1_bias, l0b1_conv2_w, l0b1_conv2_bias, l0b1_conv3_w, l0b1_conv3_bias, l0b2_conv1_w, l0b2_conv1_bias, l0b2_conv2_w, l0b2_conv2_bias, l0b2_conv3_w, l0b2_conv3_bias, l1b0_conv1_w, l1b0_conv1_bias, l1b0_conv2_w, l1b0_conv2_bias, l1b0_conv3_w, l1b0_conv3_bias, l1b0_down_w, l1b0_down_bias, l1b1_conv1_w, l1b1_conv1_bias, l1b1_conv2_w, l1b1_conv2_bias, l1b1_conv3_w, l1b1_conv3_bias, l1b2_conv1_w, l1b2_conv1_bias, l1b2_conv2_w, l1b2_conv2_bias, l1b2_conv3_w, l1b2_conv3_bias, l1b3_conv1_w, l1b3_conv1_bias, l1b3_conv2_w, l1b3_conv2_bias, l1b3_conv3_w, l1b3_conv3_bias, l2b0_conv1_w, l2b0_conv1_bias, l2b0_conv2_w, l2b0_conv2_bias, l2b0_conv3_w, l2b0_conv3_bias, l2b0_down_w, l2b0_down_bias, l2b1_conv1_w, l2b1_conv1_bias, l2b1_conv2_w, l2b1_conv2_bias, l2b1_conv3_w, l2b1_conv3_bias, l2b2_conv1_w, l2b2_conv1_bias, l2b2_conv2_w, l2b2_conv2_bias, l2b2_conv3_w, l2b2_conv3_bias, l2b3_conv1_w, l2b3_conv1_bias, l2b3_conv2_w, l2b3_conv2_bias, l2b3_conv3_w, l2b3_conv3_bias, l2b4_conv1_w, l2b4_conv1_bias, l2b4_conv2_w, l2b4_conv2_bias, l2b4_conv3_w, l2b4_conv3_bias, l2b5_conv1_w, l2b5_conv1_bias, l2b5_conv2_w, l2b5_conv2_bias, l2b5_conv3_w, l2b5_conv3_bias, x):
    xh = jnp.transpose(x, (0, 2, 3, 1)).astype(_BF16)
    xh = jnp.pad(xh, ((0, 0), (0, 0), (0, 0), (0, 1)))
    y = _stem_pool(xh, stem_w, stem_bias)

    y = _layer(y, [
        (l0b0_conv1_w, l0b0_conv1_bias, l0b0_conv2_w, l0b0_conv2_bias,
         l0b0_conv3_w, l0b0_conv3_bias, l0b0_down_w, l0b0_down_bias, 1),
        (l0b1_conv1_w, l0b1_conv1_bias, l0b1_conv2_w, l0b1_conv2_bias,
         l0b1_conv3_w, l0b1_conv3_bias, 1),
        (l0b2_conv1_w, l0b2_conv1_bias, l0b2_conv2_w, l0b2_conv2_bias,
         l0b2_conv3_w, l0b2_conv3_bias, 1),
    ])
    y = _layer(y, [
        (l1b0_conv1_w, l1b0_conv1_bias, l1b0_conv2_w, l1b0_conv2_bias,
         l1b0_conv3_w, l1b0_conv3_bias, l1b0_down_w, l1b0_down_bias, 2),
        (l1b1_conv1_w, l1b1_conv1_bias, l1b1_conv2_w, l1b1_conv2_bias,
         l1b1_conv3_w, l1b1_conv3_bias, 1),
        (l1b2_conv1_w, l1b2_conv1_bias, l1b2_conv2_w, l1b2_conv2_bias,
         l1b2_conv3_w, l1b2_conv3_bias, 1),
        (l1b3_conv1_w, l1b3_conv1_bias, l1b3_conv2_w, l1b3_conv2_bias,
         l1b3_conv3_w, l1b3_conv3_bias, 1),
    ])
    y = _layer(y, [
        (l2b0_conv1_w, l2b0_conv1_bias, l2b0_conv2_w, l2b0_conv2_bias,
         l2b0_conv3_w, l2b0_conv3_bias, l2b0_down_w, l2b0_down_bias, 1),
        (l2b1_conv1_w, l2b1_conv1_bias, l2b1_conv2_w, l2b1_conv2_bias,
         l2b1_conv3_w, l2b1_conv3_bias, 1),
        (l2b2_conv1_w, l2b2_conv1_bias, l2b2_conv2_w, l2b2_conv2_bias,
         l2b2_conv3_w, l2b2_conv3_bias, 1),
        (l2b3_conv1_w, l2b3_conv1_bias, l2b3_conv2_w, l2b3_conv2_bias,
         l2b3_conv3_w, l2b3_conv3_bias, 1),
        (l2b4_conv1_w, l2b4_conv1_bias, l2b4_conv2_w, l2b4_conv2_bias,
         l2b4_conv3_w, l2b4_conv3_bias, 1),
        (l2b5_conv1_w, l2b5_conv1_bias, l2b5_conv2_w, l2b5_conv2_bias,
         l2b5_conv3_w, l2b5_conv3_bias, 1),
    ])
    return jnp.transpose(y, (0, 3, 1, 2)).astype(jnp.float32)


# speedup vs baseline: 1.0341x; 1.0341x over previous
import functools

import jax
import jax.numpy as jnp
from jax.experimental import pallas as pl
from jax.experimental.pallas import tpu as pltpu

_BF16 = jnp.bfloat16
_VMEM_LIMIT = 48 * 1024 * 1024


def _cparams():
    return pltpu.CompilerParams(dimension_semantics=("parallel",),
                                vmem_limit_bytes=_VMEM_LIMIT)


def _evenrows(v, off, n):
    return v[off:off + 2 * n].reshape(n, 2, v.shape[1], v.shape[2])[:, 0]


def _colsplit(v):
    r, c, ch = v.shape
    v2 = v.reshape(r, c // 2, 2, ch)
    return v2[:, :, 0], v2[:, :, 1]


def _bneck_compute(cur, wr, stride, has_down, mid_ref, pk_ref):
    H, W, cin = cur.shape
    w1, b1, w2, b2, w3, b3 = wr[:6]
    mid = w1.shape[-1]
    ho, wo = H // stride, W // stride
    M = ho * wo

    h1 = jnp.dot(cur.reshape(H * W, cin), w1[...],
                 preferred_element_type=jnp.float32) + b1[...]
    mid_ref[1:H + 1, 0:W, :] = (
        jnp.maximum(h1, 0.0).astype(_BF16).reshape(H, W, mid))
    zr = jnp.zeros((1, W + 2, mid), _BF16)
    mid_ref[0:1, 0:W + 2] = zr
    mid_ref[H + 1:H + 2, 0:W + 2] = zr
    zc = jnp.zeros((H + 2, 1, mid), _BF16)
    mid_ref[0:H + 2, W:W + 1] = zc
    mp = mid_ref[0:H + 2, 0:W + 2, :]

    acc2 = jnp.zeros((M, mid), jnp.float32)
    if stride == 1:
        pk_ref[0:H + 2, 0:1, 0:mid] = jnp.zeros((H + 2, 1, mid), _BF16)
        pk_ref[0:H + 2, 1:W, 0:mid] = mp[:, 0:W - 1]
        pk_ref[0:H + 2, 0:W, mid:2 * mid] = mp[:, 0:W]
        pk_ref[0:H + 2, 0:W, 2 * mid:3 * mid] = mp[:, 1:W + 1]
        for ki in range(3):
            a = pk_ref[ki:ki + H, 0:W, :].reshape(M, 3 * mid)
            acc2 = acc2 + jnp.dot(a, w2[ki * 3 * mid:(ki + 1) * 3 * mid, :],
                                  preferred_element_type=jnp.float32)
    else:
        ce, co = _colsplit(mp)
        pk_ref[0:H + 2, 0:1, 0:mid] = jnp.zeros((H + 2, 1, mid), _BF16)
        pk_ref[0:H + 2, 1:wo, 0:mid] = co[:, 0:wo - 1]
        pk_ref[0:H + 2, 0:wo, mid:2 * mid] = ce[:, 0:wo]
        pk_ref[0:H + 2, 0:wo, 2 * mid:3 * mid] = co[:, 0:wo]
        for ki in range(3):
            a = pk_ref[ki:ki + 2 * ho, 0:wo, :]
            a = a.reshape(ho, 2, wo, 3 * mid)[:, 0].reshape(M, 3 * mid)
            acc2 = acc2 + jnp.dot(a, w2[ki * 3 * mid:(ki + 1) * 3 * mid, :],
                                  preferred_element_type=jnp.float32)
    h2 = jnp.maximum(acc2 + b2[...], 0.0).astype(_BF16)

    h3 = jnp.dot(h2, w3[...], preferred_element_type=jnp.float32) + b3[...]
    if has_down:
        wd, bd = wr[6:8]
        if stride == 1:
            xs = cur
        else:
            xe, _ = _colsplit(cur)
            xs = _evenrows(xe, 0, ho)
        idn = jnp.dot(xs.reshape(M, cin), wd[...],
                      preferred_element_type=jnp.float32) + bd[...]
        idn = idn.astype(_BF16).astype(jnp.float32)
    else:
        idn = cur.reshape(M, cin).astype(jnp.float32)
    out = jnp.maximum(h3 + idn, 0.0)
    return out.astype(_BF16).reshape(ho, wo, out.shape[-1])


def _layer_kernel(*refs, cfg, nw):
    x_ref = refs[0]
    wrefs = refs[1:1 + nw]
    o_ref = refs[1 + nw]
    mid_ref, pk_ref = refs[2 + nw:]
    cur = x_ref[0]
    i = 0
    for stride, has_down in cfg:
        k = 8 if has_down else 6
        cur = _bneck_compute(cur, wrefs[i:i + k], stride, has_down,
                             mid_ref, pk_ref)
        i += k
    o_ref[0] = cur


def _layer(x, blocks):
    N, H, W, cin0 = x.shape
    f32 = jnp.float32
    args = [x]
    in_specs = [pl.BlockSpec((1, H, W, cin0), lambda n: (n, 0, 0, 0))]
    cfg = []
    mid = blocks[0][2].shape[2]
    stride0 = blocks[0][-1]
    hl, wl = H // stride0, W // stride0
    cout = None
    for bp in blocks:
        stride = bp[-1]
        ws = bp[:-1]
        has_down = len(ws) == 8
        cfg.append((stride, has_down))
        w1, b1, w2, b2, w3, b3 = ws[:6]
        cin = w1.shape[2]
        cout = w3.shape[3]
        flat = [w1.reshape(cin, mid), b1.reshape(1, mid).astype(f32),
                w2.reshape(9 * mid, mid), b2.reshape(1, mid).astype(f32),
                w3.reshape(mid, cout), b3.reshape(1, cout).astype(f32)]
        if has_down:
            wd, bd = ws[6:8]
            flat += [wd.reshape(cin, cout), bd.reshape(1, cout).astype(f32)]
        for a in flat:
            args.append(a)
            in_specs.append(
                pl.BlockSpec(a.shape, lambda n, nd=a.ndim: (0,) * nd))
    nw = len(args) - 1
    scratch = [pltpu.VMEM((H + 2, W + 2, mid), _BF16),
               pltpu.VMEM((H + 2, wl, 3 * mid), _BF16)]
    return pl.pallas_call(
        functools.partial(_layer_kernel, cfg=tuple(cfg), nw=nw),
        out_shape=jax.ShapeDtypeStruct((N, hl, wl, cout), _BF16),
        grid=(N,),
        in_specs=in_specs,
        out_specs=pl.BlockSpec((1, hl, wl, cout), lambda n: (n, 0, 0, 0)),
        scratch_shapes=scratch,
        compiler_params=_cparams(),
    )(*args)


def _stem_kernel(x2_ref, w_ref, b_ref, o_ref, cv_ref, pk_ref, *, ho):
    x2 = x2_ref[0]
    for kj in range(4):
        pk_ref[0:ho + 3, 0:ho, kj * 16:(kj + 1) * 16] = x2[:, kj:kj + ho]
    acc = jnp.zeros((ho * ho, 64), jnp.float32)
    for ki in range(4):
        a = pk_ref[ki:ki + ho, 0:ho, :].reshape(ho * ho, 64)
        acc = acc + jnp.dot(a, w_ref[ki * 64:(ki + 1) * 64, :],
                            preferred_element_type=jnp.float32)
    h = acc + b_ref[...]
    cv_ref[1:ho + 1, 0:ho, :] = (
        jnp.maximum(h, 0.0).astype(_BF16).reshape(ho, ho, 64))
    zr = jnp.zeros((1, ho + 2, 64), _BF16)
    cv_ref[0:1] = zr
    cv_ref[ho + 1:ho + 2] = zr
    cv = cv_ref[...]
    hp = ho // 2
    cv2 = cv.reshape(hp + 1, 2, ho + 2, 64)
    re = cv2[:, 0]
    ro = cv2[:, 1]
    rm = jnp.maximum(jnp.maximum(re[0:hp], ro[0:hp]), re[1:hp + 1])
    ce, co = _colsplit(rm)
    co_sh = jnp.concatenate(
        [jnp.zeros((hp, 1, 64), _BF16), co[:, 0:hp - 1]], axis=1)
    m = jnp.maximum(jnp.maximum(ce[:, 0:hp], co[:, 0:hp]), co_sh)
    o_ref[0] = m


def _stem_pool(x_nhwc4, stem_w, stem_bias):
    N, H, _, C = x_nhwc4.shape
    ho = H // 2
    hb = ho + 3
    xpad = jnp.pad(x_nhwc4, ((0, 0), (3, 3), (3, 3), (0, 0)))
    x2 = xpad.reshape(N, hb, 2, hb, 2, C).transpose(0, 1, 3, 2, 4, 5)
    x2 = x2.reshape(N, hb, hb, 4 * C)
    wp = jnp.pad(stem_w, ((0, 1), (0, 1), (0, 0), (0, 0)))
    w = wp.reshape(4, 2, 4, 2, C, 64).transpose(0, 2, 1, 3, 4, 5)
    w = w.reshape(16 * C * 4, 64)
    return pl.pallas_call(
        functools.partial(_stem_kernel, ho=ho),
        out_shape=jax.ShapeDtypeStruct((N, ho // 2, ho // 2, 64), _BF16),
        grid=(N,),
        in_specs=[
            pl.BlockSpec((1, hb, hb, 4 * C), lambda n: (n, 0, 0, 0)),
            pl.BlockSpec((16 * C * 4, 64), lambda n: (0, 0)),
            pl.BlockSpec((1, 64), lambda n: (0, 0)),
        ],
        out_specs=pl.BlockSpec((1, ho // 2, ho // 2, 64), lambda n: (n, 0, 0, 0)),
        scratch_shapes=[pltpu.VMEM((ho + 2, ho + 2, 64), _BF16),
                        pltpu.VMEM((ho + 3, ho, 16 * C), _BF16)],
        compiler_params=_cparams(),
    )(x2, w, stem_bias.reshape(1, 64).astype(jnp.float32))


def kernel(stem_w, stem_bias, l0b0_conv1_w, l0b0_conv1_bias, l0b0_conv2_w, l0b0_conv2_bias, l0b0_conv3_w, l0b0_conv3_bias, l0b0_down_w, l0b0_down_bias, l0b1_conv1_w, l0b1_conv1_bias, l0b1_conv2_w, l0b1_conv2_bias, l0b1_conv3_w, l0b1_conv3_bias, l0b2_conv1_w, l0b2_conv1_bias, l0b2_conv2_w, l0b2_conv2_bias, l0b2_conv3_w, l0b2_conv3_bias, l1b0_conv1_w, l1b0_conv1_bias, l1b0_conv2_w, l1b0_conv2_bias, l1b0_conv3_w, l1b0_conv3_bias, l1b0_down_w, l1b0_down_bias, l1b1_conv1_w, l1b1_conv1_bias, l1b1_conv2_w, l1b1_conv2_bias, l1b1_conv3_w, l1b1_conv3_bias, l1b2_conv1_w, l1b2_conv1_bias, l1b2_conv2_w, l1b2_conv2_bias, l1b2_conv3_w, l1b2_conv3_bias, l1b3_conv1_w, l1b3_conv1_bias, l1b3_conv2_w, l1b3_conv2_bias, l1b3_conv3_w, l1b3_conv3_bias, l2b0_conv1_w, l2b0_conv1_bias, l2b0_conv2_w, l2b0_conv2_bias, l2b0_conv3_w, l2b0_conv3_bias, l2b0_down_w, l2b0_down_bias, l2b1_conv1_w, l2b1_conv1_bias, l2b1_conv2_w, l2b1_conv2_bias, l2b1_conv3_w, l2b1_conv3_bias, l2b2_conv1_w, l2b2_conv1_bias, l2b2_conv2_w, l2b2_conv2_bias, l2b2_conv3_w, l2b2_conv3_bias, l2b3_conv1_w, l2b3_conv1_bias, l2b3_conv2_w, l2b3_conv2_bias, l2b3_conv3_w, l2b3_conv3_bias, l2b4_conv1_w, l2b4_conv1_bias, l2b4_conv2_w, l2b4_conv2_bias, l2b4_conv3_w, l2b4_conv3_bias, l2b5_conv1_w, l2b5_conv1_bias, l2b5_conv2_w, l2b5_conv2_bias, l2b5_conv3_w, l2b5_conv3_bias, x):
    xh = jnp.transpose(x, (0, 2, 3, 1)).astype(_BF16)
    xh = jnp.pad(xh, ((0, 0), (0, 0), (0, 0), (0, 1)))
    y = _stem_pool(xh, stem_w, stem_bias)

    y = _layer(y, [
        (l0b0_conv1_w, l0b0_conv1_bias, l0b0_conv2_w, l0b0_conv2_bias,
         l0b0_conv3_w, l0b0_conv3_bias, l0b0_down_w, l0b0_down_bias, 1),
        (l0b1_conv1_w, l0b1_conv1_bias, l0b1_conv2_w, l0b1_conv2_bias,
         l0b1_conv3_w, l0b1_conv3_bias, 1),
        (l0b2_conv1_w, l0b2_conv1_bias, l0b2_conv2_w, l0b2_conv2_bias,
         l0b2_conv3_w, l0b2_conv3_bias, 1),
    ])
    y = _layer(y, [
        (l1b0_conv1_w, l1b0_conv1_bias, l1b0_conv2_w, l1b0_conv2_bias,
         l1b0_conv3_w, l1b0_conv3_bias, l1b0_down_w, l1b0_down_bias, 2),
        (l1b1_conv1_w, l1b1_conv1_bias, l1b1_conv2_w, l1b1_conv2_bias,
         l1b1_conv3_w, l1b1_conv3_bias, 1),
        (l1b2_conv1_w, l1b2_conv1_bias, l1b2_conv2_w, l1b2_conv2_bias,
         l1b2_conv3_w, l1b2_conv3_bias, 1),
        (l1b3_conv1_w, l1b3_conv1_bias, l1b3_conv2_w, l1b3_conv2_bias,
         l1b3_conv3_w, l1b3_conv3_bias, 1),
    ])
    y = _layer(y, [
        (l2b0_conv1_w, l2b0_conv1_bias, l2b0_conv2_w, l2b0_conv2_bias,
         l2b0_conv3_w, l2b0_conv3_bias, l2b0_down_w, l2b0_down_bias, 1),
        (l2b1_conv1_w, l2b1_conv1_bias, l2b1_conv2_w, l2b1_conv2_bias,
         l2b1_conv3_w, l2b1_conv3_bias, 1),
        (l2b2_conv1_w, l2b2_conv1_bias, l2b2_conv2_w, l2b2_conv2_bias,
         l2b2_conv3_w, l2b2_conv3_bias, 1),
        (l2b3_conv1_w, l2b3_conv1_bias, l2b3_conv2_w, l2b3_conv2_bias,
         l2b3_conv3_w, l2b3_conv3_bias, 1),
        (l2b4_conv1_w, l2b4_conv1_bias, l2b4_conv2_w, l2b4_conv2_bias,
         l2b4_conv3_w, l2b4_conv3_bias, 1),
        (l2b5_conv1_w, l2b5_conv1_bias, l2b5_conv2_w, l2b5_conv2_bias,
         l2b5_conv3_w, l2b5_conv3_bias, 1),
    ])
    return jnp.transpose(y, (0, 3, 1, 2)).astype(jnp.float32)


# speedup vs baseline: 1.0403x; 1.0060x over previous
import functools

import jax
import jax.numpy as jnp
from jax.experimental import pallas as pl
from jax.experimental.pallas import tpu as pltpu

_BF16 = jnp.bfloat16
_VMEM_LIMIT = 48 * 1024 * 1024


def _cparams():
    return pltpu.CompilerParams(dimension_semantics=("parallel",),
                                vmem_limit_bytes=_VMEM_LIMIT)


def _evenrows(v, off, n):
    return v[off:off + 2 * n].reshape(n, 2, v.shape[1], v.shape[2])[:, 0]


def _colsplit(v):
    r, c, ch = v.shape
    v2 = v.reshape(r, c // 2, 2, ch)
    return v2[:, :, 0], v2[:, :, 1]


def _bneck_compute(cur, wr, stride, has_down, mid_ref, pk_ref, tout=False):
    H, W, cin = cur.shape
    w1, b1, w2, b2, w3, b3 = wr[:6]
    mid = w1.shape[-1]
    ho, wo = H // stride, W // stride
    M = ho * wo

    h1 = jnp.dot(cur.reshape(H * W, cin), w1[...],
                 preferred_element_type=jnp.float32) + b1[...]
    mid_ref[1:H + 1, 0:W, :] = (
        jnp.maximum(h1, 0.0).astype(_BF16).reshape(H, W, mid))
    zr = jnp.zeros((1, W + 2, mid), _BF16)
    mid_ref[0:1, 0:W + 2] = zr
    mid_ref[H + 1:H + 2, 0:W + 2] = zr
    zc = jnp.zeros((H + 2, 1, mid), _BF16)
    mid_ref[0:H + 2, W:W + 1] = zc
    mp = mid_ref[0:H + 2, 0:W + 2, :]

    acc2 = jnp.zeros((M, mid), jnp.float32)
    if stride == 1:
        pk_ref[0:H + 2, 0:1, 0:mid] = jnp.zeros((H + 2, 1, mid), _BF16)
        pk_ref[0:H + 2, 1:W, 0:mid] = mp[:, 0:W - 1]
        pk_ref[0:H + 2, 0:W, mid:2 * mid] = mp[:, 0:W]
        pk_ref[0:H + 2, 0:W, 2 * mid:3 * mid] = mp[:, 1:W + 1]
        for ki in range(3):
            a = pk_ref[ki:ki + H, 0:W, :].reshape(M, 3 * mid)
            acc2 = acc2 + jnp.dot(a, w2[ki * 3 * mid:(ki + 1) * 3 * mid, :],
                                  preferred_element_type=jnp.float32)
    else:
        ce, co = _colsplit(mp)
        pk_ref[0:H + 2, 0:1, 0:mid] = jnp.zeros((H + 2, 1, mid), _BF16)
        pk_ref[0:H + 2, 1:wo, 0:mid] = co[:, 0:wo - 1]
        pk_ref[0:H + 2, 0:wo, mid:2 * mid] = ce[:, 0:wo]
        pk_ref[0:H + 2, 0:wo, 2 * mid:3 * mid] = co[:, 0:wo]
        for ki in range(3):
            a = pk_ref[ki:ki + 2 * ho, 0:wo, :]
            a = a.reshape(ho, 2, wo, 3 * mid)[:, 0].reshape(M, 3 * mid)
            acc2 = acc2 + jnp.dot(a, w2[ki * 3 * mid:(ki + 1) * 3 * mid, :],
                                  preferred_element_type=jnp.float32)
    h2 = jnp.maximum(acc2 + b2[...], 0.0).astype(_BF16)

    if tout:
        h3 = jnp.dot(w3[...], h2.T, preferred_element_type=jnp.float32) + b3[...]
        idn = cur.reshape(M, cur.shape[-1]).T.astype(jnp.float32)
        out = jnp.maximum(h3 + idn, 0.0)
        return out.astype(_BF16).astype(jnp.float32)

    h3 = jnp.dot(h2, w3[...], preferred_element_type=jnp.float32) + b3[...]
    if has_down:
        wd, bd = wr[6:8]
        if stride == 1:
            xs = cur
        else:
            xe, _ = _colsplit(cur)
            xs = _evenrows(xe, 0, ho)
        idn = jnp.dot(xs.reshape(M, cin), wd[...],
                      preferred_element_type=jnp.float32) + bd[...]
        idn = idn.astype(_BF16).astype(jnp.float32)
    else:
        idn = cur.reshape(M, cin).astype(jnp.float32)
    out = jnp.maximum(h3 + idn, 0.0)
    return out.astype(_BF16).reshape(ho, wo, out.shape[-1])


def _layer_kernel(*refs, cfg, nw):
    x_ref = refs[0]
    wrefs = refs[1:1 + nw]
    o_ref = refs[1 + nw]
    mid_ref, pk_ref = refs[2 + nw:]
    cur = x_ref[0]
    i = 0
    for stride, has_down, tout in cfg:
        k = 8 if has_down else 6
        cur = _bneck_compute(cur, wrefs[i:i + k], stride, has_down,
                             mid_ref, pk_ref, tout=tout)
        i += k
    o_ref[0] = cur


def _layer(x, blocks, tout_last=False):
    N, H, W, cin0 = x.shape
    f32 = jnp.float32
    args = [x]
    in_specs = [pl.BlockSpec((1, H, W, cin0), lambda n: (n, 0, 0, 0))]
    cfg = []
    mid = blocks[0][2].shape[2]
    stride0 = blocks[0][-1]
    hl, wl = H // stride0, W // stride0
    cout = None
    for bi, bp in enumerate(blocks):
        stride = bp[-1]
        ws = bp[:-1]
        has_down = len(ws) == 8
        tout = tout_last and bi == len(blocks) - 1
        cfg.append((stride, has_down, tout))
        w1, b1, w2, b2, w3, b3 = ws[:6]
        cin = w1.shape[2]
        cout = w3.shape[3]
        if tout:
            w3f = w3.reshape(mid, cout).T
            b3f = b3.reshape(cout, 1).astype(f32)
        else:
            w3f = w3.reshape(mid, cout)
            b3f = b3.reshape(1, cout).astype(f32)
        flat = [w1.reshape(cin, mid), b1.reshape(1, mid).astype(f32),
                w2.reshape(9 * mid, mid), b2.reshape(1, mid).astype(f32),
                w3f, b3f]
        if has_down:
            wd, bd = ws[6:8]
            flat += [wd.reshape(cin, cout), bd.reshape(1, cout).astype(f32)]
        for a in flat:
            args.append(a)
            in_specs.append(
                pl.BlockSpec(a.shape, lambda n, nd=a.ndim: (0,) * nd))
    nw = len(args) - 1
    if tout_last:
        return pl.pallas_call(
            functools.partial(_layer_kernel, cfg=tuple(cfg), nw=nw),
            out_shape=jax.ShapeDtypeStruct((N, cout, hl * wl), f32),
            grid=(N,),
            in_specs=in_specs,
            out_specs=pl.BlockSpec((1, cout, hl * wl), lambda n: (n, 0, 0)),
            scratch_shapes=[pltpu.VMEM((H + 2, W + 2, mid), _BF16),
                            pltpu.VMEM((H + 2, wl, 3 * mid), _BF16)],
            compiler_params=_cparams(),
        )(*args)
    scratch = [pltpu.VMEM((H + 2, W + 2, mid), _BF16),
               pltpu.VMEM((H + 2, wl, 3 * mid), _BF16)]
    return pl.pallas_call(
        functools.partial(_layer_kernel, cfg=tuple(cfg), nw=nw),
        out_shape=jax.ShapeDtypeStruct((N, hl, wl, cout), _BF16),
        grid=(N,),
        in_specs=in_specs,
        out_specs=pl.BlockSpec((1, hl, wl, cout), lambda n: (n, 0, 0, 0)),
        scratch_shapes=scratch,
        compiler_params=_cparams(),
    )(*args)


def _stem_kernel(x2_ref, w_ref, b_ref, o_ref, cv_ref, pk_ref, *, ho):
    x2 = x2_ref[0]
    for kj in range(4):
        pk_ref[0:ho + 3, 0:ho, kj * 16:(kj + 1) * 16] = x2[:, kj:kj + ho]
    acc = jnp.zeros((ho * ho, 64), jnp.float32)
    for ki in range(4):
        a = pk_ref[ki:ki + ho, 0:ho, :].reshape(ho * ho, 64)
        acc = acc + jnp.dot(a, w_ref[ki * 64:(ki + 1) * 64, :],
                            preferred_element_type=jnp.float32)
    h = acc + b_ref[...]
    cv_ref[1:ho + 1, 0:ho, :] = (
        jnp.maximum(h, 0.0).astype(_BF16).reshape(ho, ho, 64))
    zr = jnp.zeros((1, ho + 2, 64), _BF16)
    cv_ref[0:1] = zr
    cv_ref[ho + 1:ho + 2] = zr
    cv = cv_ref[...]
    hp = ho // 2
    cv2 = cv.reshape(hp + 1, 2, ho + 2, 64)
    re = cv2[:, 0]
    ro = cv2[:, 1]
    rm = jnp.maximum(jnp.maximum(re[0:hp], ro[0:hp]), re[1:hp + 1])
    ce, co = _colsplit(rm)
    co_sh = jnp.concatenate(
        [jnp.zeros((hp, 1, 64), _BF16), co[:, 0:hp - 1]], axis=1)
    m = jnp.maximum(jnp.maximum(ce[:, 0:hp], co[:, 0:hp]), co_sh)
    o_ref[0] = m


def _stem_pool(x_nhwc4, stem_w, stem_bias):
    N, H, _, C = x_nhwc4.shape
    ho = H // 2
    hb = ho + 3
    xpad = jnp.pad(x_nhwc4, ((0, 0), (3, 3), (3, 3), (0, 0)))
    x2 = xpad.reshape(N, hb, 2, hb, 2, C).transpose(0, 1, 3, 2, 4, 5)
    x2 = x2.reshape(N, hb, hb, 4 * C)
    wp = jnp.pad(stem_w, ((0, 1), (0, 1), (0, 0), (0, 0)))
    w = wp.reshape(4, 2, 4, 2, C, 64).transpose(0, 2, 1, 3, 4, 5)
    w = w.reshape(16 * C * 4, 64)
    return pl.pallas_call(
        functools.partial(_stem_kernel, ho=ho),
        out_shape=jax.ShapeDtypeStruct((N, ho // 2, ho // 2, 64), _BF16),
        grid=(N,),
        in_specs=[
            pl.BlockSpec((1, hb, hb, 4 * C), lambda n: (n, 0, 0, 0)),
            pl.BlockSpec((16 * C * 4, 64), lambda n: (0, 0)),
            pl.BlockSpec((1, 64), lambda n: (0, 0)),
        ],
        out_specs=pl.BlockSpec((1, ho // 2, ho // 2, 64), lambda n: (n, 0, 0, 0)),
        scratch_shapes=[pltpu.VMEM((ho + 2, ho + 2, 64), _BF16),
                        pltpu.VMEM((ho + 3, ho, 16 * C), _BF16)],
        compiler_params=_cparams(),
    )(x2, w, stem_bias.reshape(1, 64).astype(jnp.float32))


def kernel(stem_w, stem_bias, l0b0_conv1_w, l0b0_conv1_bias, l0b0_conv2_w, l0b0_conv2_bias, l0b0_conv3_w, l0b0_conv3_bias, l0b0_down_w, l0b0_down_bias, l0b1_conv1_w, l0b1_conv1_bias, l0b1_conv2_w, l0b1_conv2_bias, l0b1_conv3_w, l0b1_conv3_bias, l0b2_conv1_w, l0b2_conv1_bias, l0b2_conv2_w, l0b2_conv2_bias, l0b2_conv3_w, l0b2_conv3_bias, l1b0_conv1_w, l1b0_conv1_bias, l1b0_conv2_w, l1b0_conv2_bias, l1b0_conv3_w, l1b0_conv3_bias, l1b0_down_w, l1b0_down_bias, l1b1_conv1_w, l1b1_conv1_bias, l1b1_conv2_w, l1b1_conv2_bias, l1b1_conv3_w, l1b1_conv3_bias, l1b2_conv1_w, l1b2_conv1_bias, l1b2_conv2_w, l1b2_conv2_bias, l1b2_conv3_w, l1b2_conv3_bias, l1b3_conv1_w, l1b3_conv1_bias, l1b3_conv2_w, l1b3_conv2_bias, l1b3_conv3_w, l1b3_conv3_bias, l2b0_conv1_w, l2b0_conv1_bias, l2b0_conv2_w, l2b0_conv2_bias, l2b0_conv3_w, l2b0_conv3_bias, l2b0_down_w, l2b0_down_bias, l2b1_conv1_w, l2b1_conv1_bias, l2b1_conv2_w, l2b1_conv2_bias, l2b1_conv3_w, l2b1_conv3_bias, l2b2_conv1_w, l2b2_conv1_bias, l2b2_conv2_w, l2b2_conv2_bias, l2b2_conv3_w, l2b2_conv3_bias, l2b3_conv1_w, l2b3_conv1_bias, l2b3_conv2_w, l2b3_conv2_bias, l2b3_conv3_w, l2b3_conv3_bias, l2b4_conv1_w, l2b4_conv1_bias, l2b4_conv2_w, l2b4_conv2_bias, l2b4_conv3_w, l2b4_conv3_bias, l2b5_conv1_w, l2b5_conv1_bias, l2b5_conv2_w, l2b5_conv2_bias, l2b5_conv3_w, l2b5_conv3_bias, x):
    xh = jnp.transpose(x.astype(_BF16), (0, 2, 3, 1))
    xh = jnp.pad(xh, ((0, 0), (0, 0), (0, 0), (0, 1)))
    y = _stem_pool(xh, stem_w, stem_bias)

    y = _layer(y, [
        (l0b0_conv1_w, l0b0_conv1_bias, l0b0_conv2_w, l0b0_conv2_bias,
         l0b0_conv3_w, l0b0_conv3_bias, l0b0_down_w, l0b0_down_bias, 1),
        (l0b1_conv1_w, l0b1_conv1_bias, l0b1_conv2_w, l0b1_conv2_bias,
         l0b1_conv3_w, l0b1_conv3_bias, 1),
        (l0b2_conv1_w, l0b2_conv1_bias, l0b2_conv2_w, l0b2_conv2_bias,
         l0b2_conv3_w, l0b2_conv3_bias, 1),
    ])
    y = _layer(y, [
        (l1b0_conv1_w, l1b0_conv1_bias, l1b0_conv2_w, l1b0_conv2_bias,
         l1b0_conv3_w, l1b0_conv3_bias, l1b0_down_w, l1b0_down_bias, 2),
        (l1b1_conv1_w, l1b1_conv1_bias, l1b1_conv2_w, l1b1_conv2_bias,
         l1b1_conv3_w, l1b1_conv3_bias, 1),
        (l1b2_conv1_w, l1b2_conv1_bias, l1b2_conv2_w, l1b2_conv2_bias,
         l1b2_conv3_w, l1b2_conv3_bias, 1),
        (l1b3_conv1_w, l1b3_conv1_bias, l1b3_conv2_w, l1b3_conv2_bias,
         l1b3_conv3_w, l1b3_conv3_bias, 1),
    ])
    y = _layer(y, [
        (l2b0_conv1_w, l2b0_conv1_bias, l2b0_conv2_w, l2b0_conv2_bias,
         l2b0_conv3_w, l2b0_conv3_bias, l2b0_down_w, l2b0_down_bias, 1),
        (l2b1_conv1_w, l2b1_conv1_bias, l2b1_conv2_w, l2b1_conv2_bias,
         l2b1_conv3_w, l2b1_conv3_bias, 1),
        (l2b2_conv1_w, l2b2_conv1_bias, l2b2_conv2_w, l2b2_conv2_bias,
         l2b2_conv3_w, l2b2_conv3_bias, 1),
        (l2b3_conv1_w, l2b3_conv1_bias, l2b3_conv2_w, l2b3_conv2_bias,
         l2b3_conv3_w, l2b3_conv3_bias, 1),
        (l2b4_conv1_w, l2b4_conv1_bias, l2b4_conv2_w, l2b4_conv2_bias,
         l2b4_conv3_w, l2b4_conv3_bias, 1),
        (l2b5_conv1_w, l2b5_conv1_bias, l2b5_conv2_w, l2b5_conv2_bias,
         l2b5_conv3_w, l2b5_conv3_bias, 1),
    ], tout_last=True)
    n, c, hw = y.shape
    hh = int(hw ** 0.5)
    return y.reshape(n, c, hh, hh)
